# Initial kernel scaffold; baseline (speedup 1.0000x reference)
import functools

import jax
import jax.numpy as jnp
from jax import lax
from jax.experimental import pallas as pl
from jax.experimental.pallas import tpu as pltpu
from jax.experimental.pallas import tpu_sc as plsc

D = 128
H = 256
N_NODES = 10000
N_EDGES = 320000
TE = 2 * N_EDGES

NC = 2
NS = 16
NW = NC * NS

CHUNK = 80
PER_W = TE // NW
G_CHUNKS = PER_W // CHUNK
S_CHUNKS_TOTAL = TE // CHUNK
S_PER_W = S_CHUNKS_TOTAL // NW
NPT = N_NODES // NS

EPS = 1e-5


def _sc_gather(node_rep, node_idx):
    mesh = plsc.VectorSubcoreMesh(core_axis_name="c", subcore_axis_name="s")

    @functools.partial(
        pl.kernel,
        out_type=jax.ShapeDtypeStruct((TE, D), jnp.float32),
        mesh=mesh,
        scratch_types=[
            pltpu.VMEM((CHUNK,), jnp.int32),
            pltpu.VMEM((CHUNK, D), jnp.float32),
            pltpu.SemaphoreType.DMA,
        ],
    )
    def k(table_hbm, idx_hbm, out_hbm, idx_v, rows_v, sem):
        wid = lax.axis_index("s") * NC + lax.axis_index("c")
        base = wid * PER_W

        @pl.loop(0, G_CHUNKS)
        def _(j):
            off = base + j * CHUNK
            pltpu.sync_copy(idx_hbm.at[pl.ds(off, CHUNK)], idx_v)
            pltpu.async_copy(table_hbm.at[idx_v], rows_v, sem).wait()
            pltpu.sync_copy(rows_v, out_hbm.at[pl.ds(off, CHUNK)])

    return k(node_rep, node_idx)


def _sc_scatter_add(rows, idx2d, zeros):
    mesh = plsc.VectorSubcoreMesh(core_axis_name="c", subcore_axis_name="s")

    @functools.partial(
        pl.kernel,
        out_type=jax.ShapeDtypeStruct((NC, N_NODES, D), jnp.float32),
        mesh=mesh,
        scratch_types=[
            pltpu.VMEM((CHUNK,), jnp.int32),
            pltpu.VMEM((CHUNK, D), jnp.float32),
            pltpu.VMEM_SHARED((N_NODES, D), jnp.float32),
        ],
    )
    def k(rows_hbm, idx_hbm, zeros_hbm, out_hbm, idx_v, rows_v, acc):
        cid = lax.axis_index("c")
        sid = lax.axis_index("s")
        pltpu.sync_copy(zeros_hbm, acc.at[pl.ds(sid * NPT, NPT)])
        plsc.subcore_barrier()

        base = (cid * NS + sid) * S_PER_W

        @pl.loop(0, S_PER_W)
        def _(j):
            ch = base + j
            pltpu.sync_copy(rows_hbm.at[pl.ds(ch * CHUNK, CHUNK)], rows_v)
            pltpu.sync_copy(idx_hbm.at[ch], idx_v)
            pltpu.sync_copy(rows_v, acc.at[idx_v], add=True)

        plsc.subcore_barrier()
        pltpu.sync_copy(acc.at[pl.ds(sid * NPT, NPT)],
                        out_hbm.at[cid].at[pl.ds(sid * NPT, NPT)])

    return k(rows, idx2d, zeros)


TILE = 512
GRID_E = N_EDGES // TILE


def _e1_body(er_ref, g2_ref, w_ref, y1_ref, st_ref):
    i = pl.program_id(0)
    er = er_ref[...]
    g2 = g2_ref[...]
    w = w_ref[...]
    xl = jnp.concatenate([er[:, :D], g2[:, :D], g2[:, D:]], axis=1)
    xr = jnp.concatenate([er[:, D:], g2[:, D:], g2[:, :D]], axis=1)
    yl = jnp.dot(xl, w, preferred_element_type=jnp.float32)
    yr = jnp.dot(xr, w, preferred_element_type=jnp.float32)
    y = jnp.concatenate([yl, yr], axis=1)
    y1_ref[...] = y

    @pl.when(i == 0)
    def _():
        st_ref[...] = jnp.zeros_like(st_ref)

    st_ref[0:1, :] += jnp.sum(y, axis=0, keepdims=True)
    st_ref[1:2, :] += jnp.sum(y * y, axis=0, keepdims=True)


def _e2_body(y1_ref, st_ref, g1_ref, b1_ref, w2_ref, y2_ref, st2_ref):
    st = st_ref[...]
    cs = st[0:1, :]
    cs2 = st[1:2, :]
    m = (cs[:, :H] + cs[:, H:]) / TE
    ex2 = (cs2[:, :H] + cs2[:, H:]) / TE
    v = ex2 - m * m
    s = g1_ref[...] * lax.rsqrt(v + EPS)
    t = b1_ref[...] - m * s
    y = y1_ref[...]
    al = jnp.maximum(y[:, :H] * s + t, 0.0)
    ar = jnp.maximum(y[:, H:] * s + t, 0.0)
    w2 = w2_ref[...]
    zl = jnp.dot(al, w2, preferred_element_type=jnp.float32)
    zr = jnp.dot(ar, w2, preferred_element_type=jnp.float32)
    z = jnp.concatenate([zl, zr], axis=1)
    y2_ref[...] = z

    i = pl.program_id(0)

    @pl.when(i == 0)
    def _():
        st2_ref[...] = jnp.zeros_like(st2_ref)

    st2_ref[0:1, :] += jnp.sum(z, axis=0, keepdims=True)
    st2_ref[1:2, :] += jnp.sum(z * z, axis=0, keepdims=True)


def _e3_body(y2_ref, st2_ref, g2_ref, b2_ref, eo_ref):
    st = st2_ref[...]
    cs = st[0:1, :]
    cs2 = st[1:2, :]
    m = (cs[:, :D] + cs[:, D:]) / TE
    ex2 = (cs2[:, :D] + cs2[:, D:]) / TE
    v = ex2 - m * m
    s = g2_ref[...] * lax.rsqrt(v + EPS)
    t = b2_ref[...] - m * s
    y = y2_ref[...]
    ol = jnp.maximum(y[:, :D] * s + t, 0.0)
    orr = jnp.maximum(y[:, D:] * s + t, 0.0)
    eo_ref[...] = jnp.concatenate([ol, orr], axis=1)


NTILE = 1000
GRID_N = N_NODES // NTILE


def _n1_body(nr_ref, p0_ref, p1_ref, w1_ref, y_ref, st_ref):
    x = jnp.concatenate([nr_ref[...], p0_ref[...] + p1_ref[...]], axis=1)
    y = jnp.dot(x, w1_ref[...], preferred_element_type=jnp.float32)
    y_ref[...] = y
    i = pl.program_id(0)

    @pl.when(i == 0)
    def _():
        st_ref[...] = jnp.zeros_like(st_ref)

    st_ref[0:1, :] += jnp.sum(y, axis=0, keepdims=True)
    st_ref[1:2, :] += jnp.sum(y * y, axis=0, keepdims=True)


def _n2_body(y_ref, st_ref, g_ref, b_ref, w2_ref, z_ref, st2_ref):
    st = st_ref[...]
    m = st[0:1, :] / N_NODES
    v = st[1:2, :] / N_NODES - m * m
    s = g_ref[...] * lax.rsqrt(v + EPS)
    t = b_ref[...] - m * s
    a = jnp.maximum(y_ref[...] * s + t, 0.0)
    z = jnp.dot(a, w2_ref[...], preferred_element_type=jnp.float32)
    z_ref[...] = z
    i = pl.program_id(0)

    @pl.when(i == 0)
    def _():
        st2_ref[...] = jnp.zeros_like(st2_ref)

    st2_ref[0:1, :] += jnp.sum(z, axis=0, keepdims=True)
    st2_ref[1:2, :] += jnp.sum(z * z, axis=0, keepdims=True)


def _n3_body(z_ref, st2_ref, g_ref, b_ref, o_ref):
    st = st2_ref[...]
    m = st[0:1, :] / N_NODES
    v = st[1:2, :] / N_NODES - m * m
    s = g_ref[...] * lax.rsqrt(v + EPS)
    t = b_ref[...] - m * s
    o_ref[...] = jnp.maximum(z_ref[...] * s + t, 0.0)


def _row_spec(tile, width):
    return pl.BlockSpec((tile, width), lambda i: (i, 0))


def _full_spec(shape):
    return pl.BlockSpec(shape, lambda i: tuple(0 for _ in shape))


def kernel(node_rep, edge_rep, edge_index,
           edge_W1, edge_g1, edge_b1, edge_W2, edge_g2, edge_b2,
           node_W1, node_g1, node_b1, node_W2, node_g2, node_b2):
    f32 = jnp.float32
    node_idx = edge_index.T.reshape(-1).astype(jnp.int32)

    gathered = _sc_gather(node_rep, node_idx)
    g2 = gathered.reshape(N_EDGES, 2 * D)
    er = edge_rep.reshape(N_EDGES, 2 * D)

    wstack = jnp.concatenate(
        [edge_W1[:D], edge_W1[D:2 * D] + edge_W1[2 * D:], edge_W1[2 * D:]],
        axis=0)

    y1, st1 = pl.pallas_call(
        _e1_body,
        grid=(GRID_E,),
        in_specs=[_row_spec(TILE, 2 * D), _row_spec(TILE, 2 * D),
                  _full_spec((3 * D, H))],
        out_specs=[_row_spec(TILE, 2 * H), _full_spec((2, 2 * H))],
        out_shape=[jax.ShapeDtypeStruct((N_EDGES, 2 * H), f32),
                   jax.ShapeDtypeStruct((2, 2 * H), f32)],
    )(er, g2, wstack)

    y2, st2 = pl.pallas_call(
        _e2_body,
        grid=(GRID_E,),
        in_specs=[_row_spec(TILE, 2 * H), _full_spec((2, 2 * H)),
                  _full_spec((1, H)), _full_spec((1, H)),
                  _full_spec((H, D))],
        out_specs=[_row_spec(TILE, 2 * D), _full_spec((2, 2 * D))],
        out_shape=[jax.ShapeDtypeStruct((N_EDGES, 2 * D), f32),
                   jax.ShapeDtypeStruct((2, 2 * D), f32)],
    )(y1, st1, edge_g1.reshape(1, H), edge_b1.reshape(1, H), edge_W2)

    eo_pair = pl.pallas_call(
        _e3_body,
        grid=(GRID_E,),
        in_specs=[_row_spec(TILE, 2 * D), _full_spec((2, 2 * D)),
                  _full_spec((1, D)), _full_spec((1, D))],
        out_specs=_row_spec(TILE, 2 * D),
        out_shape=jax.ShapeDtypeStruct((N_EDGES, 2 * D), f32),
    )(y2, st2, edge_g2.reshape(1, D), edge_b2.reshape(1, D))

    edge_out = eo_pair.reshape(TE, D)

    idx2d = node_idx.reshape(S_CHUNKS_TOTAL, CHUNK)
    zeros = jnp.zeros((NPT, D), f32)
    partials = _sc_scatter_add(edge_out, idx2d, zeros)

    yn, stn1 = pl.pallas_call(
        _n1_body,
        grid=(GRID_N,),
        in_specs=[_row_spec(NTILE, D), _row_spec(NTILE, D),
                  _row_spec(NTILE, D), _full_spec((2 * D, H))],
        out_specs=[_row_spec(NTILE, H), _full_spec((2, H))],
        out_shape=[jax.ShapeDtypeStruct((N_NODES, H), f32),
                   jax.ShapeDtypeStruct((2, H), f32)],
    )(node_rep, partials[0], partials[1], node_W1)

    zn, stn2 = pl.pallas_call(
        _n2_body,
        grid=(GRID_N,),
        in_specs=[_row_spec(NTILE, H), _full_spec((2, H)),
                  _full_spec((1, H)), _full_spec((1, H)),
                  _full_spec((H, D))],
        out_specs=[_row_spec(NTILE, D), _full_spec((2, D))],
        out_shape=[jax.ShapeDtypeStruct((N_NODES, D), f32),
                   jax.ShapeDtypeStruct((2, D), f32)],
    )(yn, stn1, node_g1.reshape(1, H), node_b1.reshape(1, H), node_W2)

    node_out = pl.pallas_call(
        _n3_body,
        grid=(GRID_N,),
        in_specs=[_row_spec(NTILE, D), _full_spec((2, D)),
                  _full_spec((1, D)), _full_spec((1, D))],
        out_specs=_row_spec(NTILE, D),
        out_shape=jax.ShapeDtypeStruct((N_NODES, D), f32),
    )(zn, stn2, node_g2.reshape(1, D), node_b2.reshape(1, D))

    return (node_out, edge_out)


# speedup vs baseline: 2.2393x; 2.2393x over previous
import functools

import jax
import jax.numpy as jnp
from jax import lax
from jax.experimental import pallas as pl
from jax.experimental.pallas import tpu as pltpu
from jax.experimental.pallas import tpu_sc as plsc

D = 128
H = 256
N_NODES = 10000
N_EDGES = 320000
TE = 2 * N_EDGES

NC = 2
NS = 16
NW = NC * NS

CHUNK = 80
PER_W = TE // NW
G_CHUNKS = PER_W // CHUNK
S_CHUNKS_TOTAL = TE // CHUNK
S_PER_W = S_CHUNKS_TOTAL // NW
N_PAD = 10240
NPT = N_PAD // NS

EPS = 1e-5


def _sc_gather(node_rep, node_idx):
    mesh = plsc.VectorSubcoreMesh(core_axis_name="c", subcore_axis_name="s")

    @functools.partial(
        pl.kernel,
        out_type=jax.ShapeDtypeStruct((TE, D), jnp.float32),
        mesh=mesh,
        scratch_types=[
            pltpu.VMEM((CHUNK,), jnp.int32),
            pltpu.VMEM((CHUNK, D), jnp.float32),
            pltpu.SemaphoreType.DMA,
        ],
    )
    def k(table_hbm, idx_hbm, out_hbm, idx_v, rows_v, sem):
        wid = lax.axis_index("s") * NC + lax.axis_index("c")
        base = wid * PER_W

        @pl.loop(0, G_CHUNKS)
        def _(j):
            off = base + j * CHUNK
            pltpu.sync_copy(idx_hbm.at[pl.ds(off, CHUNK)], idx_v)
            pltpu.async_copy(table_hbm.at[idx_v], rows_v, sem).wait()
            pltpu.sync_copy(rows_v, out_hbm.at[pl.ds(off, CHUNK)])

    return k(node_rep, node_idx)


def _sc_scatter_add(rows, idx2d, zeros):
    mesh = plsc.VectorSubcoreMesh(core_axis_name="c", subcore_axis_name="s")

    @functools.partial(
        pl.kernel,
        out_type=jax.ShapeDtypeStruct((NC, N_PAD, D), jnp.float32),
        mesh=mesh,
        scratch_types=[
            pltpu.VMEM((CHUNK,), jnp.int32),
            pltpu.VMEM((CHUNK, D), jnp.float32),
            pltpu.VMEM_SHARED((N_PAD, D), jnp.float32),
        ],
    )
    def k(rows_hbm, idx_hbm, zeros_hbm, out_hbm, idx_v, rows_v, acc):
        cid = lax.axis_index("c")
        sid = lax.axis_index("s")
        pltpu.sync_copy(zeros_hbm, acc.at[pl.ds(sid * NPT, NPT)])
        plsc.subcore_barrier()

        base = (cid * NS + sid) * S_PER_W

        @pl.loop(0, S_PER_W)
        def _(j):
            ch = base + j
            pltpu.sync_copy(rows_hbm.at[pl.ds(ch * CHUNK, CHUNK)], rows_v)
            pltpu.sync_copy(idx_hbm.at[ch], idx_v)
            pltpu.sync_copy(rows_v, acc.at[idx_v], add=True)

        plsc.subcore_barrier()
        pltpu.sync_copy(acc.at[pl.ds(sid * NPT, NPT)],
                        out_hbm.at[cid].at[pl.ds(sid * NPT, NPT)])

    return k(rows, idx2d, zeros)


TILE = 512
GRID_E = N_EDGES // TILE


def _e1_body(er_ref, g2_ref, w_ref, y1_ref, st_ref):
    i = pl.program_id(0)
    er = er_ref[...]
    g2 = g2_ref[...]
    w = w_ref[...]
    xl = jnp.concatenate([er[:, :D], g2[:, :D], g2[:, D:]], axis=1)
    xr = jnp.concatenate([er[:, D:], g2[:, D:], g2[:, :D]], axis=1)
    yl = jnp.dot(xl, w, preferred_element_type=jnp.float32)
    yr = jnp.dot(xr, w, preferred_element_type=jnp.float32)
    y = jnp.concatenate([yl, yr], axis=1)
    y1_ref[...] = y

    @pl.when(i == 0)
    def _():
        st_ref[...] = jnp.zeros_like(st_ref)

    st_ref[0:1, :] += jnp.sum(y, axis=0, keepdims=True)
    st_ref[1:2, :] += jnp.sum(y * y, axis=0, keepdims=True)


def _e2_body(y1_ref, st_ref, g1_ref, b1_ref, w2_ref, y2_ref, st2_ref):
    st = st_ref[...]
    cs = st[0:1, :]
    cs2 = st[1:2, :]
    m = (cs[:, :H] + cs[:, H:]) / TE
    ex2 = (cs2[:, :H] + cs2[:, H:]) / TE
    v = ex2 - m * m
    s = g1_ref[...] * lax.rsqrt(v + EPS)
    t = b1_ref[...] - m * s
    y = y1_ref[...]
    al = jnp.maximum(y[:, :H] * s + t, 0.0)
    ar = jnp.maximum(y[:, H:] * s + t, 0.0)
    w2 = w2_ref[...]
    zl = jnp.dot(al, w2, preferred_element_type=jnp.float32)
    zr = jnp.dot(ar, w2, preferred_element_type=jnp.float32)
    z = jnp.concatenate([zl, zr], axis=1)
    y2_ref[...] = z

    i = pl.program_id(0)

    @pl.when(i == 0)
    def _():
        st2_ref[...] = jnp.zeros_like(st2_ref)

    st2_ref[0:1, :] += jnp.sum(z, axis=0, keepdims=True)
    st2_ref[1:2, :] += jnp.sum(z * z, axis=0, keepdims=True)


def _e3_body(y2_ref, st2_ref, g2_ref, b2_ref, eo_ref):
    st = st2_ref[...]
    cs = st[0:1, :]
    cs2 = st[1:2, :]
    m = (cs[:, :D] + cs[:, D:]) / TE
    ex2 = (cs2[:, :D] + cs2[:, D:]) / TE
    v = ex2 - m * m
    s = g2_ref[...] * lax.rsqrt(v + EPS)
    t = b2_ref[...] - m * s
    y = y2_ref[...]
    ol = jnp.maximum(y[:, :D] * s + t, 0.0)
    orr = jnp.maximum(y[:, D:] * s + t, 0.0)
    eo_ref[...] = jnp.concatenate([ol, orr], axis=1)


NTILE = 1000
GRID_N = N_NODES // NTILE


def _n1_body(nr_ref, p0_ref, p1_ref, w1_ref, y_ref, st_ref):
    x = jnp.concatenate([nr_ref[...], p0_ref[...] + p1_ref[...]], axis=1)
    y = jnp.dot(x, w1_ref[...], preferred_element_type=jnp.float32)
    y_ref[...] = y
    i = pl.program_id(0)

    @pl.when(i == 0)
    def _():
        st_ref[...] = jnp.zeros_like(st_ref)

    st_ref[0:1, :] += jnp.sum(y, axis=0, keepdims=True)
    st_ref[1:2, :] += jnp.sum(y * y, axis=0, keepdims=True)


def _n2_body(y_ref, st_ref, g_ref, b_ref, w2_ref, z_ref, st2_ref):
    st = st_ref[...]
    m = st[0:1, :] / N_NODES
    v = st[1:2, :] / N_NODES - m * m
    s = g_ref[...] * lax.rsqrt(v + EPS)
    t = b_ref[...] - m * s
    a = jnp.maximum(y_ref[...] * s + t, 0.0)
    z = jnp.dot(a, w2_ref[...], preferred_element_type=jnp.float32)
    z_ref[...] = z
    i = pl.program_id(0)

    @pl.when(i == 0)
    def _():
        st2_ref[...] = jnp.zeros_like(st2_ref)

    st2_ref[0:1, :] += jnp.sum(z, axis=0, keepdims=True)
    st2_ref[1:2, :] += jnp.sum(z * z, axis=0, keepdims=True)


def _n3_body(z_ref, st2_ref, g_ref, b_ref, o_ref):
    st = st2_ref[...]
    m = st[0:1, :] / N_NODES
    v = st[1:2, :] / N_NODES - m * m
    s = g_ref[...] * lax.rsqrt(v + EPS)
    t = b_ref[...] - m * s
    o_ref[...] = jnp.maximum(z_ref[...] * s + t, 0.0)


def _row_spec(tile, width):
    return pl.BlockSpec((tile, width), lambda i: (i, 0))


def _full_spec(shape):
    return pl.BlockSpec(shape, lambda i: tuple(0 for _ in shape))


def kernel(node_rep, edge_rep, edge_index,
           edge_W1, edge_g1, edge_b1, edge_W2, edge_g2, edge_b2,
           node_W1, node_g1, node_b1, node_W2, node_g2, node_b2):
    f32 = jnp.float32
    node_idx = edge_index.T.reshape(-1).astype(jnp.int32)

    gathered = _sc_gather(node_rep, node_idx)
    g2 = gathered.reshape(N_EDGES, 2 * D)
    er = edge_rep.reshape(N_EDGES, 2 * D)

    wstack = jnp.concatenate(
        [edge_W1[:D], edge_W1[D:2 * D] + edge_W1[2 * D:], edge_W1[2 * D:]],
        axis=0)

    y1, st1 = pl.pallas_call(
        _e1_body,
        grid=(GRID_E,),
        in_specs=[_row_spec(TILE, 2 * D), _row_spec(TILE, 2 * D),
                  _full_spec((3 * D, H))],
        out_specs=[_row_spec(TILE, 2 * H), _full_spec((2, 2 * H))],
        out_shape=[jax.ShapeDtypeStruct((N_EDGES, 2 * H), f32),
                   jax.ShapeDtypeStruct((2, 2 * H), f32)],
    )(er, g2, wstack)

    y2, st2 = pl.pallas_call(
        _e2_body,
        grid=(GRID_E,),
        in_specs=[_row_spec(TILE, 2 * H), _full_spec((2, 2 * H)),
                  _full_spec((1, H)), _full_spec((1, H)),
                  _full_spec((H, D))],
        out_specs=[_row_spec(TILE, 2 * D), _full_spec((2, 2 * D))],
        out_shape=[jax.ShapeDtypeStruct((N_EDGES, 2 * D), f32),
                   jax.ShapeDtypeStruct((2, 2 * D), f32)],
    )(y1, st1, edge_g1.reshape(1, H), edge_b1.reshape(1, H), edge_W2)

    eo_pair = pl.pallas_call(
        _e3_body,
        grid=(GRID_E,),
        in_specs=[_row_spec(TILE, 2 * D), _full_spec((2, 2 * D)),
                  _full_spec((1, D)), _full_spec((1, D))],
        out_specs=_row_spec(TILE, 2 * D),
        out_shape=jax.ShapeDtypeStruct((N_EDGES, 2 * D), f32),
    )(y2, st2, edge_g2.reshape(1, D), edge_b2.reshape(1, D))

    edge_out = eo_pair.reshape(TE, D)

    idx2d = node_idx.reshape(S_CHUNKS_TOTAL, CHUNK)
    zeros = jnp.zeros((NPT, D), f32)
    partials = _sc_scatter_add(edge_out, idx2d, zeros)[:, :N_NODES]

    yn, stn1 = pl.pallas_call(
        _n1_body,
        grid=(GRID_N,),
        in_specs=[_row_spec(NTILE, D), _row_spec(NTILE, D),
                  _row_spec(NTILE, D), _full_spec((2 * D, H))],
        out_specs=[_row_spec(NTILE, H), _full_spec((2, H))],
        out_shape=[jax.ShapeDtypeStruct((N_NODES, H), f32),
                   jax.ShapeDtypeStruct((2, H), f32)],
    )(node_rep, partials[0], partials[1], node_W1)

    zn, stn2 = pl.pallas_call(
        _n2_body,
        grid=(GRID_N,),
        in_specs=[_row_spec(NTILE, H), _full_spec((2, H)),
                  _full_spec((1, H)), _full_spec((1, H)),
                  _full_spec((H, D))],
        out_specs=[_row_spec(NTILE, D), _full_spec((2, D))],
        out_shape=[jax.ShapeDtypeStruct((N_NODES, D), f32),
                   jax.ShapeDtypeStruct((2, D), f32)],
    )(yn, stn1, node_g1.reshape(1, H), node_b1.reshape(1, H), node_W2)

    node_out = pl.pallas_call(
        _n3_body,
        grid=(GRID_N,),
        in_specs=[_row_spec(NTILE, D), _full_spec((2, D)),
                  _full_spec((1, D)), _full_spec((1, D))],
        out_specs=_row_spec(NTILE, D),
        out_shape=jax.ShapeDtypeStruct((N_NODES, D), f32),
    )(zn, stn2, node_g2.reshape(1, D), node_b2.reshape(1, D))

    return (node_out, edge_out)


# speedup vs baseline: 2.3796x; 1.0627x over previous
import functools

import jax
import jax.numpy as jnp
from jax import lax
from jax.experimental import pallas as pl
from jax.experimental.pallas import tpu as pltpu
from jax.experimental.pallas import tpu_sc as plsc

D = 128
H = 256
N_NODES = 10000
N_EDGES = 320000
TE = 2 * N_EDGES

NC = 2
NS = 16
NW = NC * NS

CHUNK = 80
PER_W = TE // NW
G_CHUNKS = PER_W // CHUNK
S_CHUNKS_TOTAL = TE // CHUNK
S_PER_W = S_CHUNKS_TOTAL // NW
N_PAD = 10240
NPT = N_PAD // NS

EPS = 1e-5


def _sc_gather(table, node_idx):
    mesh = plsc.VectorSubcoreMesh(core_axis_name="c", subcore_axis_name="s")
    dg = table.shape[1]

    @functools.partial(
        pl.kernel,
        out_type=jax.ShapeDtypeStruct((TE, dg), table.dtype),
        mesh=mesh,
        scratch_types=[
            pltpu.VMEM((CHUNK,), jnp.int32),
            pltpu.VMEM((CHUNK, dg), table.dtype),
            pltpu.SemaphoreType.DMA,
        ],
    )
    def k(table_hbm, idx_hbm, out_hbm, idx_v, rows_v, sem):
        wid = lax.axis_index("s") * NC + lax.axis_index("c")
        base = wid * PER_W

        @pl.loop(0, G_CHUNKS)
        def _(j):
            off = base + j * CHUNK
            pltpu.sync_copy(idx_hbm.at[pl.ds(off, CHUNK)], idx_v)
            pltpu.async_copy(table_hbm.at[idx_v], rows_v, sem).wait()
            pltpu.sync_copy(rows_v, out_hbm.at[pl.ds(off, CHUNK)])

    return k(table, node_idx)


def _sc_scatter_add(rows, idx2d, zeros):
    mesh = plsc.VectorSubcoreMesh(core_axis_name="c", subcore_axis_name="s")

    @functools.partial(
        pl.kernel,
        out_type=jax.ShapeDtypeStruct((NC, N_PAD, D), jnp.float32),
        mesh=mesh,
        scratch_types=[
            pltpu.VMEM((CHUNK,), jnp.int32),
            pltpu.VMEM((CHUNK, D), jnp.float32),
            pltpu.VMEM_SHARED((N_PAD, D), jnp.float32),
        ],
    )
    def k(rows_hbm, idx_hbm, zeros_hbm, out_hbm, idx_v, rows_v, acc):
        cid = lax.axis_index("c")
        sid = lax.axis_index("s")
        pltpu.sync_copy(zeros_hbm, acc.at[pl.ds(sid * NPT, NPT)])
        plsc.subcore_barrier()

        base = (cid * NS + sid) * S_PER_W

        @pl.loop(0, S_PER_W)
        def _(j):
            ch = base + j
            pltpu.sync_copy(rows_hbm.at[pl.ds(ch * CHUNK, CHUNK)], rows_v)
            pltpu.sync_copy(idx_hbm.at[ch], idx_v)
            pltpu.sync_copy(rows_v, acc.at[idx_v], add=True)

        plsc.subcore_barrier()
        pltpu.sync_copy(acc.at[pl.ds(sid * NPT, NPT)],
                        out_hbm.at[cid].at[pl.ds(sid * NPT, NPT)])

    return k(rows, idx2d, zeros)


TILE = 512
GRID_E = N_EDGES // TILE


def _e1_body(er_ref, g2_ref, w_ref, y1_ref, st_ref):
    i = pl.program_id(0)
    er = er_ref[...]
    g2 = g2_ref[...].astype(jnp.bfloat16)
    w = w_ref[...]
    xl = jnp.concatenate([er[:, :D], g2[:, :D], g2[:, D:]], axis=1)
    xr = jnp.concatenate([er[:, D:], g2[:, D:], g2[:, :D]], axis=1)
    yl = jnp.dot(xl, w, preferred_element_type=jnp.float32)
    yr = jnp.dot(xr, w, preferred_element_type=jnp.float32)
    y = jnp.concatenate([yl, yr], axis=1)
    y1_ref[...] = y.astype(jnp.bfloat16)

    @pl.when(i == 0)
    def _():
        st_ref[...] = jnp.zeros_like(st_ref)

    st_ref[0:1, :] += jnp.sum(y, axis=0, keepdims=True)
    st_ref[1:2, :] += jnp.sum(y * y, axis=0, keepdims=True)


def _e2_body(y1_ref, st_ref, g1_ref, b1_ref, w2_ref, y2_ref, st2_ref):
    st = st_ref[...]
    cs = st[0:1, :]
    cs2 = st[1:2, :]
    m = (cs[:, :H] + cs[:, H:]) / TE
    ex2 = (cs2[:, :H] + cs2[:, H:]) / TE
    v = ex2 - m * m
    s = g1_ref[...] * lax.rsqrt(v + EPS)
    t = b1_ref[...] - m * s
    y = y1_ref[...].astype(jnp.float32)
    al = jnp.maximum(y[:, :H] * s + t, 0.0).astype(jnp.bfloat16)
    ar = jnp.maximum(y[:, H:] * s + t, 0.0).astype(jnp.bfloat16)
    w2 = w2_ref[...]
    zl = jnp.dot(al, w2, preferred_element_type=jnp.float32)
    zr = jnp.dot(ar, w2, preferred_element_type=jnp.float32)
    z = jnp.concatenate([zl, zr], axis=1)
    y2_ref[...] = z.astype(jnp.bfloat16)

    i = pl.program_id(0)

    @pl.when(i == 0)
    def _():
        st2_ref[...] = jnp.zeros_like(st2_ref)

    st2_ref[0:1, :] += jnp.sum(z, axis=0, keepdims=True)
    st2_ref[1:2, :] += jnp.sum(z * z, axis=0, keepdims=True)


def _e3_body(y2_ref, st2_ref, g2_ref, b2_ref, eo_ref):
    st = st2_ref[...]
    cs = st[0:1, :]
    cs2 = st[1:2, :]
    m = (cs[:, :D] + cs[:, D:]) / TE
    ex2 = (cs2[:, :D] + cs2[:, D:]) / TE
    v = ex2 - m * m
    s = g2_ref[...] * lax.rsqrt(v + EPS)
    t = b2_ref[...] - m * s
    y = y2_ref[...].astype(jnp.float32)
    ol = jnp.maximum(y[:, :D] * s + t, 0.0)
    orr = jnp.maximum(y[:, D:] * s + t, 0.0)
    eo_ref[...] = jnp.concatenate([ol, orr], axis=1)


NTILE = 1000
GRID_N = N_NODES // NTILE


def _n1_body(nr_ref, p0_ref, p1_ref, w1_ref, y_ref, st_ref):
    x = jnp.concatenate([nr_ref[...], p0_ref[...] + p1_ref[...]], axis=1)
    y = jnp.dot(x, w1_ref[...], preferred_element_type=jnp.float32)
    y_ref[...] = y
    i = pl.program_id(0)

    @pl.when(i == 0)
    def _():
        st_ref[...] = jnp.zeros_like(st_ref)

    st_ref[0:1, :] += jnp.sum(y, axis=0, keepdims=True)
    st_ref[1:2, :] += jnp.sum(y * y, axis=0, keepdims=True)


def _n2_body(y_ref, st_ref, g_ref, b_ref, w2_ref, z_ref, st2_ref):
    st = st_ref[...]
    m = st[0:1, :] / N_NODES
    v = st[1:2, :] / N_NODES - m * m
    s = g_ref[...] * lax.rsqrt(v + EPS)
    t = b_ref[...] - m * s
    a = jnp.maximum(y_ref[...] * s + t, 0.0)
    z = jnp.dot(a, w2_ref[...], preferred_element_type=jnp.float32)
    z_ref[...] = z
    i = pl.program_id(0)

    @pl.when(i == 0)
    def _():
        st2_ref[...] = jnp.zeros_like(st2_ref)

    st2_ref[0:1, :] += jnp.sum(z, axis=0, keepdims=True)
    st2_ref[1:2, :] += jnp.sum(z * z, axis=0, keepdims=True)


def _n3_body(z_ref, st2_ref, g_ref, b_ref, o_ref):
    st = st2_ref[...]
    m = st[0:1, :] / N_NODES
    v = st[1:2, :] / N_NODES - m * m
    s = g_ref[...] * lax.rsqrt(v + EPS)
    t = b_ref[...] - m * s
    o_ref[...] = jnp.maximum(z_ref[...] * s + t, 0.0)


def _row_spec(tile, width):
    return pl.BlockSpec((tile, width), lambda i: (i, 0))


def _full_spec(shape):
    return pl.BlockSpec(shape, lambda i: tuple(0 for _ in shape))


def kernel(node_rep, edge_rep, edge_index,
           edge_W1, edge_g1, edge_b1, edge_W2, edge_g2, edge_b2,
           node_W1, node_g1, node_b1, node_W2, node_g2, node_b2):
    f32 = jnp.float32
    bf16 = jnp.bfloat16
    node_idx = edge_index.T.reshape(-1).astype(jnp.int32)

    gathered = _sc_gather(node_rep, node_idx)
    g2 = gathered.reshape(N_EDGES, 2 * D)
    er = edge_rep.astype(bf16).reshape(N_EDGES, 2 * D)

    wstack = jnp.concatenate(
        [edge_W1[:D], edge_W1[D:2 * D] + edge_W1[2 * D:], edge_W1[2 * D:]],
        axis=0).astype(bf16)

    y1, st1 = pl.pallas_call(
        _e1_body,
        grid=(GRID_E,),
        in_specs=[_row_spec(TILE, 2 * D), _row_spec(TILE, 2 * D),
                  _full_spec((3 * D, H))],
        out_specs=[_row_spec(TILE, 2 * H), _full_spec((2, 2 * H))],
        out_shape=[jax.ShapeDtypeStruct((N_EDGES, 2 * H), bf16),
                   jax.ShapeDtypeStruct((2, 2 * H), f32)],
    )(er, g2, wstack)

    y2, st2 = pl.pallas_call(
        _e2_body,
        grid=(GRID_E,),
        in_specs=[_row_spec(TILE, 2 * H), _full_spec((2, 2 * H)),
                  _full_spec((1, H)), _full_spec((1, H)),
                  _full_spec((H, D))],
        out_specs=[_row_spec(TILE, 2 * D), _full_spec((2, 2 * D))],
        out_shape=[jax.ShapeDtypeStruct((N_EDGES, 2 * D), bf16),
                   jax.ShapeDtypeStruct((2, 2 * D), f32)],
    )(y1, st1, edge_g1.reshape(1, H), edge_b1.reshape(1, H),
      edge_W2.astype(bf16))

    eo_pair = pl.pallas_call(
        _e3_body,
        grid=(GRID_E,),
        in_specs=[_row_spec(TILE, 2 * D), _full_spec((2, 2 * D)),
                  _full_spec((1, D)), _full_spec((1, D))],
        out_specs=_row_spec(TILE, 2 * D),
        out_shape=jax.ShapeDtypeStruct((N_EDGES, 2 * D), f32),
    )(y2, st2, edge_g2.reshape(1, D), edge_b2.reshape(1, D))

    edge_out = eo_pair.reshape(TE, D)

    idx2d = node_idx.reshape(S_CHUNKS_TOTAL, CHUNK)
    zeros = jnp.zeros((NPT, D), f32)
    partials = _sc_scatter_add(edge_out, idx2d, zeros)[:, :N_NODES]

    yn, stn1 = pl.pallas_call(
        _n1_body,
        grid=(GRID_N,),
        in_specs=[_row_spec(NTILE, D), _row_spec(NTILE, D),
                  _row_spec(NTILE, D), _full_spec((2 * D, H))],
        out_specs=[_row_spec(NTILE, H), _full_spec((2, H))],
        out_shape=[jax.ShapeDtypeStruct((N_NODES, H), f32),
                   jax.ShapeDtypeStruct((2, H), f32)],
    )(node_rep, partials[0], partials[1], node_W1)

    zn, stn2 = pl.pallas_call(
        _n2_body,
        grid=(GRID_N,),
        in_specs=[_row_spec(NTILE, H), _full_spec((2, H)),
                  _full_spec((1, H)), _full_spec((1, H)),
                  _full_spec((H, D))],
        out_specs=[_row_spec(NTILE, D), _full_spec((2, D))],
        out_shape=[jax.ShapeDtypeStruct((N_NODES, D), f32),
                   jax.ShapeDtypeStruct((2, D), f32)],
    )(yn, stn1, node_g1.reshape(1, H), node_b1.reshape(1, H), node_W2)

    node_out = pl.pallas_call(
        _n3_body,
        grid=(GRID_N,),
        in_specs=[_row_spec(NTILE, D), _full_spec((2, D)),
                  _full_spec((1, D)), _full_spec((1, D))],
        out_specs=_row_spec(NTILE, D),
        out_shape=jax.ShapeDtypeStruct((N_NODES, D), f32),
    )(zn, stn2, node_g2.reshape(1, D), node_b2.reshape(1, D))

    return (node_out, edge_out)
